# baseline (speedup 1.0000x reference)
import functools
import jax
import jax.numpy as jnp
from jax import lax
from jax.experimental import pallas as pl
from jax.experimental.pallas import tpu as pltpu
from jax.experimental.pallas import tpu_sc as plsc

B = 16384
D = 64
L = 16
CH = 256
RB = 512


def _repack_body(x_ref, o_ref):
    o_ref[:, 0:D] = x_ref[0:RB, :]
    o_ref[:, D:2 * D] = x_ref[RB:2 * RB, :]


def _repack(table):
    n = table.shape[0]
    grid = (n + 2 * RB - 1) // (2 * RB)
    return pl.pallas_call(
        _repack_body,
        grid=(grid,),
        in_specs=[pl.BlockSpec((2 * RB, D), lambda i: (i, 0))],
        out_specs=pl.BlockSpec((RB, 2 * D), lambda i: (i, 0)),
        out_shape=jax.ShapeDtypeStruct((grid * RB, 2 * D), jnp.float32),
    )(table)


@jax.jit
def _bprmf_gather(user, pos_item, neg_item, embed_user, embed_item):
    dense_u = _repack(embed_user)
    dense_i = _repack(embed_item)

    info = plsc.get_sparse_core_info()
    nc, ns = info.num_cores, info.num_subcores
    nw = nc * ns
    bpw = B // nw
    mesh = plsc.VectorSubcoreMesh(core_axis_name="c", subcore_axis_name="s")

    @functools.partial(
        pl.kernel,
        mesh=mesh,
        out_type=(
            jax.ShapeDtypeStruct((B, D), jnp.float32),
            jax.ShapeDtypeStruct((B, D), jnp.float32),
            jax.ShapeDtypeStruct((B, D), jnp.float32),
        ),
        scratch_types=[
            pltpu.VMEM((bpw,), jnp.int32),
            pltpu.VMEM((bpw,), jnp.int32),
            pltpu.VMEM((CH, 2 * D), jnp.float32),
            pltpu.VMEM((CH, D), jnp.float32),
            pltpu.SemaphoreType.DMA,
        ],
    )
    def gather(user_hbm, pos_hbm, neg_hbm, du_hbm, di_hbm,
               out_u, out_p, out_n, idx_v, fix_v, buf, outb, sem):
        wid = lax.axis_index("s") * nc + lax.axis_index("c")
        base = wid * bpw

        def one_table(idx_hbm, dn_hbm, out_hbm):
            pltpu.sync_copy(idx_hbm.at[pl.ds(base, bpw)], idx_v)

            @plsc.parallel_loop(0, bpw // L, unroll=4)
            def fix_body(m):
                v = idx_v[pl.ds(m * L, L)]
                fix_v[pl.ds(m * L, L)] = (
                    lax.shift_left(lax.shift_right_logical(v, 10), 9)
                    + (v & (RB - 1)))

            def chunk_body(c, _):
                cp = pltpu.async_copy(
                    dn_hbm.at[fix_v.at[pl.ds(c * CH, CH)]], buf, sem)
                cp.wait()

                def sel_body(g, _):
                    j0 = g * L
                    v = idx_v[pl.ds(c * CH + j0, L)]
                    off16 = (lax.shift_right_logical(v, 9) & 1) * D
                    for jj in range(L):
                        o = off16[jj]
                        for kk in range(D // L):
                            outb[j0 + jj, pl.ds(kk * L, L)] = (
                                buf[j0 + jj, pl.ds(o + kk * L, L)])
                    return _
                lax.fori_loop(0, CH // L, sel_body, 0)
                pltpu.sync_copy(outb, out_hbm.at[pl.ds(base + c * CH, CH)])
                return _
            lax.fori_loop(0, bpw // CH, chunk_body, 0)

        one_table(user_hbm, du_hbm, out_u)
        one_table(pos_hbm, di_hbm, out_p)
        one_table(neg_hbm, di_hbm, out_n)

    return gather(user, pos_item, neg_item, dense_u, dense_i)


def kernel(user, pos_item, neg_item, embed_user, embed_item):
    return _bprmf_gather(user, pos_item, neg_item, embed_user, embed_item)


# speedup vs baseline: 1.8604x; 1.8604x over previous
import functools
import jax
import jax.numpy as jnp
from jax import lax
from jax.experimental import pallas as pl
from jax.experimental.pallas import tpu as pltpu
from jax.experimental.pallas import tpu_sc as plsc

B = 16384
D = 64
L = 16
CH = 256


def _make_gather(n_idx_args, nc, ns):
    nw = nc * ns
    bpw = B // nw
    mesh = plsc.VectorSubcoreMesh(core_axis_name="c", subcore_axis_name="s")

    @functools.partial(
        pl.kernel,
        mesh=mesh,
        out_type=tuple(
            jax.ShapeDtypeStruct((B, D), jnp.float32)
            for _ in range(n_idx_args)),
        scratch_types=[
            pltpu.VMEM((bpw,), jnp.int32),
            pltpu.VMEM((bpw,), jnp.int32),
            pltpu.VMEM((CH, 2 * D), jnp.float32),
            pltpu.VMEM((CH, D), jnp.float32),
            pltpu.SemaphoreType.DMA,
        ],
    )
    def gather(*args):
        idx_hbms = args[:n_idx_args]
        tab_hbm = args[n_idx_args]
        outs = args[n_idx_args + 1:2 * n_idx_args + 1]
        idx_v, fix_v, buf, outb, sem = args[2 * n_idx_args + 1:]
        wid = lax.axis_index("s") * nc + lax.axis_index("c")
        base = wid * bpw

        def one_batch(idx_hbm, out_hbm):
            pltpu.sync_copy(idx_hbm.at[pl.ds(base, bpw)], idx_v)

            @plsc.parallel_loop(0, bpw // L, unroll=4)
            def fix_body(m):
                fix_v[pl.ds(m * L, L)] = lax.shift_right_logical(
                    idx_v[pl.ds(m * L, L)], 1)

            def chunk_body(c, _):
                cp = pltpu.async_copy(
                    tab_hbm.at[fix_v.at[pl.ds(c * CH, CH)]], buf, sem)
                cp.wait()

                def sel_body(g, _):
                    j0 = g * L
                    off16 = (idx_v[pl.ds(c * CH + j0, L)] & 1) * D
                    for jj in range(L):
                        o = off16[jj]
                        for kk in range(D // L):
                            outb[j0 + jj, pl.ds(kk * L, L)] = (
                                buf[j0 + jj, pl.ds(o + kk * L, L)])
                    return _
                lax.fori_loop(0, CH // L, sel_body, 0)
                pltpu.sync_copy(outb, out_hbm.at[pl.ds(base + c * CH, CH)])
                return _
            lax.fori_loop(0, bpw // CH, chunk_body, 0)

        for idx_hbm, out_hbm in zip(idx_hbms, outs):
            one_batch(idx_hbm, out_hbm)

    return gather


@jax.jit
def _bprmf_gather(user, pos_item, neg_item, embed_user, embed_item):
    eu2 = embed_user.reshape(embed_user.shape[0] // 2, 2 * D)
    ei2 = embed_item.reshape(embed_item.shape[0] // 2, 2 * D)

    info = plsc.get_sparse_core_info()
    nc, ns = info.num_cores, info.num_subcores
    out_u, = _make_gather(1, nc, ns)(user, eu2)
    out_p, out_n = _make_gather(2, nc, ns)(pos_item, neg_item, ei2)
    return out_u, out_p, out_n


def kernel(user, pos_item, neg_item, embed_user, embed_item):
    return _bprmf_gather(user, pos_item, neg_item, embed_user, embed_item)


# speedup vs baseline: 2.8741x; 1.5449x over previous
import functools
import jax
import jax.numpy as jnp
from jax import lax
from jax.experimental import pallas as pl
from jax.experimental.pallas import tpu as pltpu
from jax.experimental.pallas import tpu_sc as plsc

B = 16384
D = 64
L = 16
NS = 4


@jax.jit
def _bprmf_gather(user, pos_item, neg_item, embed_user, embed_item):
    info = plsc.get_sparse_core_info()
    nc, ns = info.num_cores, info.num_subcores
    nw = nc * ns
    bpw = B // nw
    mesh = plsc.VectorSubcoreMesh(core_axis_name="c", subcore_axis_name="s")

    @functools.partial(
        pl.kernel,
        mesh=mesh,
        out_type=(
            jax.ShapeDtypeStruct((B, D), jnp.float32),
            jax.ShapeDtypeStruct((B, D), jnp.float32),
            jax.ShapeDtypeStruct((B, D), jnp.float32),
        ),
        scratch_types=[
            pltpu.VMEM((bpw,), jnp.int32),
            pltpu.VMEM((bpw, D), jnp.float32),
            pltpu.SemaphoreType.DMA,
            pltpu.SemaphoreType.DMA,
            pltpu.SemaphoreType.DMA,
            pltpu.SemaphoreType.DMA,
        ],
    )
    def k(user_hbm, pos_hbm, neg_hbm, eu_hbm, ei_hbm,
          out_u, out_p, out_n, idx_v, rows_v, s0, s1, s2, s3):
        wid = lax.axis_index("s") * nc + lax.axis_index("c")
        base = wid * bpw
        sems = (s0, s1, s2, s3)

        def one_table(idx_hbm, tab_hbm, out_hbm):
            pltpu.sync_copy(idx_hbm.at[pl.ds(base, bpw)], idx_v)

            @plsc.parallel_loop(0, bpw // L, unroll=2)
            def group_body(g):
                v16 = idx_v[pl.ds(g * L, L)]
                for jj in range(L):
                    r = v16[jj]
                    pltpu.async_copy(
                        tab_hbm.at[r], rows_v.at[g * L + jj], sems[jj % NS])

            for q in range(NS):
                pltpu.make_async_copy(
                    tab_hbm.at[pl.ds(0, bpw // NS)],
                    rows_v.at[pl.ds(0, bpw // NS)], sems[q]).wait()
            pltpu.sync_copy(rows_v, out_hbm.at[pl.ds(base, bpw)])

        one_table(user_hbm, eu_hbm, out_u)
        one_table(pos_hbm, ei_hbm, out_p)
        one_table(neg_hbm, ei_hbm, out_n)

    return k(user, pos_item, neg_item, embed_user, embed_item)


def kernel(user, pos_item, neg_item, embed_user, embed_item):
    return _bprmf_gather(user, pos_item, neg_item, embed_user, embed_item)
